# Initial kernel scaffold; baseline (speedup 1.0000x reference)
import jax
import jax.numpy as jnp
from jax.experimental import pallas as pl


def kernel(feats, edge_index, key_emb, val_emb, W_self0, W_neigh0, W_self1, W_neigh1, W_cls):
    raise NotImplementedError("write your pallas kernel here")


# speedup vs baseline: 3.1701x; 3.1701x over previous
import functools

import jax
import jax.numpy as jnp
from jax import lax
from jax.experimental import pallas as pl
from jax.experimental.pallas import tpu as pltpu
from jax.experimental.pallas import tpu_sc as plsc

N = 10000
E = 320000
H = 128
OUT = 64

NC = 2
NS = 16
NW = NC * NS
CHUNK = 128

EDGE_CHUNKS_PER_W = -(-E // (NW * CHUNK))
E_PAD = EDGE_CHUNKS_PER_W * NW * CHUNK
EMB_CHUNKS_PER_W = -(-N // (NW * CHUNK))
N_PAD = EMB_CHUNKS_PER_W * NW * CHUNK
AGG_ROWS = N + 16
AGG_PER_TILE = AGG_ROWS // NS
DEG_LEN = EDGE_CHUNKS_PER_W * CHUNK
DEG_PER_TILE = DEG_LEN // NS


def _mesh():
    return plsc.VectorSubcoreMesh(
        core_axis_name="c", subcore_axis_name="s", num_cores=NC, num_subcores=NS
    )


def _sc_emb_deg_body(key_hbm, val_hbm, kidx_h, vidx_h, dst_h, zdeg_h, ones_h,
                     a_h, b_h, degp_h,
                     idx_v, rows_v, didx_v, ones_v, deg_sh, sem):
    c = lax.axis_index("c")
    s = lax.axis_index("s")
    wid = s * NC + c

    zbase = s * DEG_PER_TILE
    pltpu.sync_copy(zdeg_h.at[pl.ds(zbase, DEG_PER_TILE)],
                    deg_sh.at[pl.ds(zbase, DEG_PER_TILE)])
    pltpu.sync_copy(ones_h, ones_v)
    plsc.subcore_barrier()

    for t in range(EMB_CHUNKS_PER_W):
        row = wid * EMB_CHUNKS_PER_W + t
        out_base = row * CHUNK
        pltpu.sync_copy(kidx_h.at[row], idx_v)
        pltpu.async_copy(key_hbm.at[idx_v], rows_v, sem).wait()
        pltpu.sync_copy(rows_v, a_h.at[pl.ds(out_base, CHUNK)])
        pltpu.sync_copy(vidx_h.at[row], idx_v)
        pltpu.async_copy(val_hbm.at[idx_v], rows_v, sem).wait()
        pltpu.sync_copy(rows_v, b_h.at[pl.ds(out_base, CHUNK)])

    def deg_body(j, carry):
        row = wid * EDGE_CHUNKS_PER_W + j
        pltpu.sync_copy(dst_h.at[row], didx_v)
        pltpu.sync_copy(ones_v, deg_sh.at[didx_v], add=True)
        return carry

    lax.fori_loop(0, EDGE_CHUNKS_PER_W, deg_body, 0)
    plsc.subcore_barrier()

    pltpu.sync_copy(deg_sh.at[pl.ds(zbase, DEG_PER_TILE)],
                    degp_h.at[c, pl.ds(zbase, DEG_PER_TILE)])


@functools.partial(jax.jit, static_argnames=())
def _sc_emb_deg(key_emb, val_emb, kidx, vidx, dst2, zdeg, ones):
    kern = pl.kernel(
        _sc_emb_deg_body,
        out_type=(
            jax.ShapeDtypeStruct((N_PAD, H), jnp.float32),
            jax.ShapeDtypeStruct((N_PAD, H), jnp.float32),
            jax.ShapeDtypeStruct((NC, DEG_LEN), jnp.float32),
        ),
        mesh=_mesh(),
        scratch_types=[
            pltpu.VMEM((CHUNK,), jnp.int32),
            pltpu.VMEM((CHUNK, H), jnp.float32),
            pltpu.VMEM((CHUNK,), jnp.int32),
            pltpu.VMEM((CHUNK,), jnp.float32),
            pltpu.VMEM_SHARED((DEG_LEN,), jnp.float32),
            pltpu.SemaphoreType.DMA,
        ],
        compiler_params=pltpu.CompilerParams(use_tc_tiling_on_sc=False),
    )
    return kern(key_emb, val_emb, kidx, vidx, dst2, zdeg, ones)


def _sc_spmm_body(y_hbm, src_h, dst_h, zagg_h, aggp_h,
                  sidx_v, didx_v, rows_v, agg_sh, sem):
    c = lax.axis_index("c")
    s = lax.axis_index("s")
    wid = s * NC + c

    zbase = s * AGG_PER_TILE
    pltpu.sync_copy(zagg_h.at[pl.ds(zbase, AGG_PER_TILE)],
                    agg_sh.at[pl.ds(zbase, AGG_PER_TILE)])
    plsc.subcore_barrier()

    def body(j, carry):
        row = wid * EDGE_CHUNKS_PER_W + j
        pltpu.sync_copy(src_h.at[row], sidx_v)
        pltpu.async_copy(y_hbm.at[sidx_v], rows_v, sem).wait()
        pltpu.sync_copy(dst_h.at[row], didx_v)
        pltpu.sync_copy(rows_v, agg_sh.at[didx_v], add=True)
        return carry

    lax.fori_loop(0, EDGE_CHUNKS_PER_W, body, 0)
    plsc.subcore_barrier()

    pltpu.sync_copy(agg_sh.at[pl.ds(zbase, AGG_PER_TILE)],
                    aggp_h.at[c, pl.ds(zbase, AGG_PER_TILE)])


@jax.jit
def _sc_spmm(y, src2, dst2, zagg):
    kern = pl.kernel(
        _sc_spmm_body,
        out_type=jax.ShapeDtypeStruct((NC, AGG_ROWS, H), jnp.float32),
        mesh=_mesh(),
        scratch_types=[
            pltpu.VMEM((CHUNK,), jnp.int32),
            pltpu.VMEM((CHUNK,), jnp.int32),
            pltpu.VMEM((CHUNK, H), jnp.float32),
            pltpu.VMEM_SHARED((AGG_ROWS, H), jnp.float32),
            pltpu.SemaphoreType.DMA,
        ],
        compiler_params=pltpu.CompilerParams(use_tc_tiling_on_sc=False),
    )
    return kern(y, src2, dst2, zagg)


ROWS_BLK = 1000


def _tc1_body(a_ref, b_ref, ws_ref, wn_ref, s_ref, y_ref):
    h = jnp.maximum(a_ref[...] + b_ref[...], 0.0)
    s_ref[...] = jnp.dot(h, ws_ref[...], preferred_element_type=jnp.float32)
    y_ref[...] = jnp.dot(h, wn_ref[...], preferred_element_type=jnp.float32)


@jax.jit
def _tc1(a, b, ws, wn):
    grid = (N // ROWS_BLK,)
    blk = pl.BlockSpec((ROWS_BLK, H), lambda i: (i, 0))
    wblk = pl.BlockSpec((H, H), lambda i: (0, 0))
    return pl.pallas_call(
        _tc1_body,
        grid=grid,
        in_specs=[blk, blk, wblk, wblk],
        out_specs=[blk, blk],
        out_shape=[
            jax.ShapeDtypeStruct((N, H), jnp.float32),
            jax.ShapeDtypeStruct((N, H), jnp.float32),
        ],
    )(a, b, ws, wn)


def _tc2_body(s0_ref, a0_ref, a1_ref, d0_ref, d1_ref, ws_ref, wn_ref,
              s_ref, y_ref):
    scale = 1.0 / jnp.maximum(d0_ref[...] + d1_ref[...], 1.0)
    h = s0_ref[...] + (a0_ref[...] + a1_ref[...]) * scale
    s_ref[...] = jnp.dot(h, ws_ref[...], preferred_element_type=jnp.float32)
    y_ref[...] = jnp.dot(h, wn_ref[...], preferred_element_type=jnp.float32)


@jax.jit
def _tc2(s0, a0, a1, d0, d1, ws, wn):
    grid = (N // ROWS_BLK,)
    blk = pl.BlockSpec((ROWS_BLK, H), lambda i: (i, 0))
    dblk = pl.BlockSpec((ROWS_BLK, 1), lambda i: (i, 0))
    wblk = pl.BlockSpec((H, H), lambda i: (0, 0))
    return pl.pallas_call(
        _tc2_body,
        grid=grid,
        in_specs=[blk, blk, blk, dblk, dblk, wblk, wblk],
        out_specs=[blk, blk],
        out_shape=[
            jax.ShapeDtypeStruct((N, H), jnp.float32),
            jax.ShapeDtypeStruct((N, H), jnp.float32),
        ],
    )(s0, a0, a1, d0, d1, ws, wn)


def _tc3_body(s1_ref, a0_ref, a1_ref, d0_ref, d1_ref, wc_ref, o_ref):
    scale = 1.0 / jnp.maximum(d0_ref[...] + d1_ref[...], 1.0)
    h = s1_ref[...] + (a0_ref[...] + a1_ref[...]) * scale
    o_ref[...] = jnp.dot(h, wc_ref[...], preferred_element_type=jnp.float32)


@jax.jit
def _tc3(s1, a0, a1, d0, d1, wc):
    grid = (N // ROWS_BLK,)
    blk = pl.BlockSpec((ROWS_BLK, H), lambda i: (i, 0))
    dblk = pl.BlockSpec((ROWS_BLK, 1), lambda i: (i, 0))
    wblk = pl.BlockSpec((H, OUT), lambda i: (0, 0))
    oblk = pl.BlockSpec((ROWS_BLK, OUT), lambda i: (i, 0))
    return pl.pallas_call(
        _tc3_body,
        grid=grid,
        in_specs=[blk, blk, blk, dblk, dblk, wblk],
        out_specs=oblk,
        out_shape=jax.ShapeDtypeStruct((N, OUT), jnp.float32),
    )(s1, a0, a1, d0, d1, wc)


def kernel(feats, edge_index, key_emb, val_emb, W_self0, W_neigh0, W_self1,
           W_neigh1, W_cls):
    kidx = jnp.pad(feats[:, 0], (0, N_PAD - N)).reshape(-1, CHUNK)
    vidx = jnp.pad(feats[:, 1], (0, N_PAD - N)).reshape(-1, CHUNK)
    src2 = jnp.pad(edge_index[0], (0, E_PAD - E)).reshape(-1, CHUNK)
    dst2 = jnp.pad(edge_index[1], (0, E_PAD - E),
                   constant_values=N).reshape(-1, CHUNK)
    zdeg = jnp.zeros((DEG_LEN,), jnp.float32)
    zagg = jnp.zeros((AGG_ROWS, H), jnp.float32)
    ones = jnp.ones((CHUNK,), jnp.float32)

    a_pad, b_pad, degp = _sc_emb_deg(key_emb, val_emb, kidx, vidx, dst2,
                                     zdeg, ones)
    d0 = degp[0, :N, None]
    d1 = degp[1, :N, None]

    s0, y0 = _tc1(a_pad[:N], b_pad[:N], W_self0, W_neigh0)
    aggp0 = _sc_spmm(y0, src2, dst2, zagg)
    s1, y1 = _tc2(s0, aggp0[0, :N], aggp0[1, :N], d0, d1, W_self1, W_neigh1)
    aggp1 = _sc_spmm(y1, src2, dst2, zagg)
    out = _tc3(s1, aggp1[0, :N], aggp1[1, :N], d0, d1, W_cls)
    return out


# speedup vs baseline: 5.1187x; 1.6147x over previous
import functools

import jax
import jax.numpy as jnp
from jax import lax
from jax.experimental import pallas as pl
from jax.experimental.pallas import tpu as pltpu
from jax.experimental.pallas import tpu_sc as plsc

N = 10000
E = 320000
H = 128
HC = H // 2
OUT = 64

NC = 2
NS = 16
NW = NC * NS
CHUNK = 128

G = 2
E_CHUNKS = -(-E // CHUNK)
CPW = -(-E_CHUNKS // (NS * G)) * G
NG = CPW // G
E_PAD = CPW * NS * CHUNK

EMB_CPW = -(-N // (NW * CHUNK))
N_PAD = EMB_CPW * NW * CHUNK
AGG_ROWS = N + 16
AGG_PER_TILE = AGG_ROWS // NS
DEG_LEN = 10112
DEG_PER_TILE = DEG_LEN // NS


def _mesh():
    return plsc.VectorSubcoreMesh(
        core_axis_name="c", subcore_axis_name="s", num_cores=NC, num_subcores=NS
    )


_SC_PARAMS = pltpu.CompilerParams(use_tc_tiling_on_sc=False)


def _sc_emb_body(key_hbm, val_hbm, kidx_h, vidx_h, a_h, b_h,
                 kidx_v, vidx_v, bufs, sem_g, sem_s):
    c = lax.axis_index("c")
    s = lax.axis_index("s")
    wid = s * NC + c

    pltpu.sync_copy(kidx_h.at[pl.ds(wid * EMB_CPW, EMB_CPW)], kidx_v)
    pltpu.sync_copy(vidx_h.at[pl.ds(wid * EMB_CPW, EMB_CPW)], vidx_v)
    for t in range(EMB_CPW):
        pltpu.async_copy(key_hbm.at[kidx_v.at[t]], bufs.at[t], sem_g)
        pltpu.async_copy(val_hbm.at[vidx_v.at[t]], bufs.at[EMB_CPW + t], sem_g)
    for t in range(2 * EMB_CPW):
        pltpu.make_async_copy(key_hbm.at[pl.ds(0, CHUNK)], bufs.at[t],
                              sem_g).wait()
    for t in range(EMB_CPW):
        base = (wid * EMB_CPW + t) * CHUNK
        pltpu.async_copy(bufs.at[t], a_h.at[pl.ds(base, CHUNK)], sem_s)
        pltpu.async_copy(bufs.at[EMB_CPW + t], b_h.at[pl.ds(base, CHUNK)],
                         sem_s)
    for t in range(2 * EMB_CPW):
        pltpu.make_async_copy(bufs.at[t], key_hbm.at[pl.ds(0, CHUNK)],
                              sem_s).wait()


@jax.jit
def _sc_emb(key_emb, val_emb, kidx, vidx):
    kern = pl.kernel(
        _sc_emb_body,
        out_type=(
            jax.ShapeDtypeStruct((N_PAD, H), jnp.float32),
            jax.ShapeDtypeStruct((N_PAD, H), jnp.float32),
        ),
        mesh=_mesh(),
        scratch_types=[
            pltpu.VMEM((EMB_CPW, CHUNK), jnp.int32),
            pltpu.VMEM((EMB_CPW, CHUNK), jnp.int32),
            pltpu.VMEM((2 * EMB_CPW, CHUNK, H), jnp.float32),
            pltpu.SemaphoreType.DMA,
            pltpu.SemaphoreType.DMA,
        ],
        compiler_params=_SC_PARAMS,
    )
    return kern(key_emb, val_emb, kidx, vidx)


def _sc_spmm_body(with_deg, *refs):
    if with_deg:
        (y_hbm, src_h, dst_h, zagg_h, zdeg_h, ones_h, aggp_h, degp_h,
         sidx_v, didx_v, ones_v, bufs, agg_sh, deg_sh,
         sem_g, sem_s, sem_d) = refs
    else:
        (y_hbm, src_h, dst_h, zagg_h, aggp_h,
         sidx_v, didx_v, bufs, agg_sh,
         sem_g, sem_s) = refs
    c = lax.axis_index("c")
    s = lax.axis_index("s")

    zbase = s * AGG_PER_TILE
    pltpu.sync_copy(zagg_h.at[pl.ds(zbase, AGG_PER_TILE)],
                    agg_sh.at[pl.ds(zbase, AGG_PER_TILE)])
    if with_deg:
        dzbase = s * DEG_PER_TILE
        pltpu.sync_copy(zdeg_h.at[pl.ds(dzbase, DEG_PER_TILE)],
                        deg_sh.at[pl.ds(dzbase, DEG_PER_TILE)])
        pltpu.sync_copy(ones_h, ones_v)

    pltpu.sync_copy(src_h.at[pl.ds(s * CPW, CPW)], sidx_v)
    pltpu.sync_copy(dst_h.at[pl.ds(s * CPW, CPW)], didx_v)

    yc = y_hbm.at[c]

    def fire_gathers(g, bank):
        for t in range(G):
            pltpu.async_copy(yc.at[sidx_v.at[g * G + t]],
                             bufs.at[bank * G + t], sem_g)

    fire_gathers(0, 0)
    plsc.subcore_barrier()

    def body(g, carry):
        p = lax.rem(g, 2)
        for t in range(G):
            pltpu.make_async_copy(yc.at[pl.ds(0, CHUNK)], bufs.at[t],
                                  sem_g).wait()

        @pl.when(g > 0)
        def _():
            for t in range(G):
                pltpu.make_async_copy(yc.at[pl.ds(0, CHUNK)], bufs.at[t],
                                      sem_s).wait()

        @pl.when(g + 1 < NG)
        def _():
            fire_gathers(g + 1, 1 - p)

        for t in range(G):
            pltpu.async_copy(bufs.at[p * G + t],
                             agg_sh.at[didx_v.at[g * G + t]], sem_s, add=True)
            if with_deg:
                pltpu.async_copy(ones_v, deg_sh.at[didx_v.at[g * G + t]],
                                 sem_d, add=True)
        return carry

    lax.fori_loop(0, NG, body, 0)
    for t in range(G):
        pltpu.make_async_copy(yc.at[pl.ds(0, CHUNK)], bufs.at[t],
                              sem_s).wait()
    if with_deg:
        pltpu.make_async_copy(dst_h.at[pl.ds(0, CPW)], didx_v, sem_d).wait()
    plsc.subcore_barrier()

    pltpu.sync_copy(agg_sh.at[pl.ds(zbase, AGG_PER_TILE)],
                    aggp_h.at[c, pl.ds(zbase, AGG_PER_TILE)])
    if with_deg:
        pltpu.sync_copy(deg_sh.at[pl.ds(dzbase, DEG_PER_TILE)],
                        degp_h.at[c, pl.ds(dzbase, DEG_PER_TILE)])


def _make_spmm(with_deg):
    out_type = [jax.ShapeDtypeStruct((NC, AGG_ROWS, HC), jnp.float32)]
    scratch = [
        pltpu.VMEM((CPW, CHUNK), jnp.int32),
        pltpu.VMEM((CPW, CHUNK), jnp.int32),
    ]
    if with_deg:
        out_type.append(jax.ShapeDtypeStruct((NC, DEG_LEN), jnp.float32))
        scratch.append(pltpu.VMEM((CHUNK,), jnp.float32))
    scratch.append(pltpu.VMEM((2 * G, CHUNK, HC), jnp.float32))
    scratch.append(pltpu.VMEM_SHARED((AGG_ROWS, HC), jnp.float32))
    if with_deg:
        scratch.append(pltpu.VMEM_SHARED((DEG_LEN,), jnp.float32))
    scratch.append(pltpu.SemaphoreType.DMA)
    scratch.append(pltpu.SemaphoreType.DMA)
    if with_deg:
        scratch.append(pltpu.SemaphoreType.DMA)

    return pl.kernel(
        functools.partial(_sc_spmm_body, with_deg),
        out_type=tuple(out_type),
        mesh=_mesh(),
        scratch_types=scratch,
        compiler_params=_SC_PARAMS,
    )


@jax.jit
def _sc_spmm_deg(y, src2, dst2, zagg, zdeg, ones):
    return _make_spmm(True)(y, src2, dst2, zagg, zdeg, ones)


@jax.jit
def _sc_spmm(y, src2, dst2, zagg):
    return _make_spmm(False)(y, src2, dst2, zagg)[0]


ROWS_BLK = 1000


def _tc1_body(a_ref, b_ref, ws_ref, wn_ref, s_ref, y_ref):
    h = jnp.maximum(a_ref[...] + b_ref[...], 0.0)
    s_ref[...] = jnp.dot(h, ws_ref[...], preferred_element_type=jnp.float32)
    y = jnp.dot(h, wn_ref[...], preferred_element_type=jnp.float32)
    y_ref[0] = y[:, :HC]
    y_ref[1] = y[:, HC:]


@jax.jit
def _tc1(a, b, ws, wn):
    grid = (N // ROWS_BLK,)
    blk = pl.BlockSpec((ROWS_BLK, H), lambda i: (i, 0))
    wblk = pl.BlockSpec((H, H), lambda i: (0, 0))
    yblk = pl.BlockSpec((NC, ROWS_BLK, HC), lambda i: (0, i, 0))
    return pl.pallas_call(
        _tc1_body,
        grid=grid,
        in_specs=[blk, blk, wblk, wblk],
        out_specs=[blk, yblk],
        out_shape=[
            jax.ShapeDtypeStruct((N, H), jnp.float32),
            jax.ShapeDtypeStruct((NC, N, HC), jnp.float32),
        ],
    )(a, b, ws, wn)


def _tc2_body(s0_ref, a0_ref, a1_ref, d_ref, ws_ref, wn_ref, s_ref, y_ref):
    scale = 1.0 / jnp.maximum(d_ref[...], 1.0)
    agg = jnp.concatenate([a0_ref[...], a1_ref[...]], axis=1)
    h = s0_ref[...] + agg * scale
    s_ref[...] = jnp.dot(h, ws_ref[...], preferred_element_type=jnp.float32)
    y = jnp.dot(h, wn_ref[...], preferred_element_type=jnp.float32)
    y_ref[0] = y[:, :HC]
    y_ref[1] = y[:, HC:]


@jax.jit
def _tc2(s0, a0, a1, d, ws, wn):
    grid = (N // ROWS_BLK,)
    blk = pl.BlockSpec((ROWS_BLK, H), lambda i: (i, 0))
    hblk = pl.BlockSpec((ROWS_BLK, HC), lambda i: (i, 0))
    dblk = pl.BlockSpec((ROWS_BLK, 1), lambda i: (i, 0))
    wblk = pl.BlockSpec((H, H), lambda i: (0, 0))
    yblk = pl.BlockSpec((NC, ROWS_BLK, HC), lambda i: (0, i, 0))
    return pl.pallas_call(
        _tc2_body,
        grid=grid,
        in_specs=[blk, hblk, hblk, dblk, wblk, wblk],
        out_specs=[blk, yblk],
        out_shape=[
            jax.ShapeDtypeStruct((N, H), jnp.float32),
            jax.ShapeDtypeStruct((NC, N, HC), jnp.float32),
        ],
    )(s0, a0, a1, d, ws, wn)


def _tc3_body(s1_ref, a0_ref, a1_ref, d_ref, wc_ref, o_ref):
    scale = 1.0 / jnp.maximum(d_ref[...], 1.0)
    agg = jnp.concatenate([a0_ref[...], a1_ref[...]], axis=1)
    h = s1_ref[...] + agg * scale
    o_ref[...] = jnp.dot(h, wc_ref[...], preferred_element_type=jnp.float32)


@jax.jit
def _tc3(s1, a0, a1, d, wc):
    grid = (N // ROWS_BLK,)
    blk = pl.BlockSpec((ROWS_BLK, H), lambda i: (i, 0))
    hblk = pl.BlockSpec((ROWS_BLK, HC), lambda i: (i, 0))
    dblk = pl.BlockSpec((ROWS_BLK, 1), lambda i: (i, 0))
    wblk = pl.BlockSpec((H, OUT), lambda i: (0, 0))
    oblk = pl.BlockSpec((ROWS_BLK, OUT), lambda i: (i, 0))
    return pl.pallas_call(
        _tc3_body,
        grid=grid,
        in_specs=[blk, hblk, hblk, dblk, wblk],
        out_specs=oblk,
        out_shape=jax.ShapeDtypeStruct((N, OUT), jnp.float32),
    )(s1, a0, a1, d, wc)


def kernel(feats, edge_index, key_emb, val_emb, W_self0, W_neigh0, W_self1,
           W_neigh1, W_cls):
    kidx = jnp.pad(feats[:, 0], (0, N_PAD - N)).reshape(-1, CHUNK)
    vidx = jnp.pad(feats[:, 1], (0, N_PAD - N)).reshape(-1, CHUNK)
    src2 = jnp.pad(edge_index[0], (0, E_PAD - E)).reshape(-1, CHUNK)
    dst2 = jnp.pad(edge_index[1], (0, E_PAD - E),
                   constant_values=N).reshape(-1, CHUNK)
    zdeg = jnp.zeros((DEG_LEN,), jnp.float32)
    zagg = jnp.zeros((AGG_ROWS, HC), jnp.float32)
    ones = jnp.ones((CHUNK,), jnp.float32)

    a_pad, b_pad = _sc_emb(key_emb, val_emb, kidx, vidx)
    s0, y0 = _tc1(a_pad[:N], b_pad[:N], W_self0, W_neigh0)
    aggp0, degp = _sc_spmm_deg(y0, src2, dst2, zagg, zdeg, ones)
    d = degp[0, :N, None]
    s1, y1 = _tc2(s0, aggp0[0, :N], aggp0[1, :N], d, W_self1, W_neigh1)
    aggp1 = _sc_spmm(y1, src2, dst2, zagg)
    out = _tc3(s1, aggp1[0, :N], aggp1[1, :N], d, W_cls)
    return out


# speedup vs baseline: 5.2425x; 1.0242x over previous
import functools

import jax
import jax.numpy as jnp
from jax import lax
from jax.experimental import pallas as pl
from jax.experimental.pallas import tpu as pltpu
from jax.experimental.pallas import tpu_sc as plsc

N = 10000
E = 320000
H = 128
HC = H // 2
OUT = 64

NC = 2
NS = 16
NW = NC * NS
CHUNK = 256

E_CHUNKS = -(-E // CHUNK)
CPW = -(-E_CHUNKS // NS)
NG = CPW
E_PAD = CPW * NS * CHUNK

EMB_CHUNK = 128
EMB_CPW = -(-N // (NW * EMB_CHUNK))
N_PAD = EMB_CPW * NW * EMB_CHUNK
AGG_ROWS = N + 16
AGG_PER_TILE = AGG_ROWS // NS
DEG_LEN = 10112
DEG_PER_TILE = DEG_LEN // NS


def _mesh():
    return plsc.VectorSubcoreMesh(
        core_axis_name="c", subcore_axis_name="s", num_cores=NC, num_subcores=NS
    )


_SC_PARAMS = pltpu.CompilerParams(use_tc_tiling_on_sc=False)


def _sc_emb_body(key_hbm, val_hbm, kidx_h, vidx_h, a_h, b_h,
                 kidx_v, vidx_v, bufs, sem_g, sem_s):
    c = lax.axis_index("c")
    s = lax.axis_index("s")
    wid = s * NC + c

    pltpu.sync_copy(kidx_h.at[pl.ds(wid * EMB_CPW, EMB_CPW)], kidx_v)
    pltpu.sync_copy(vidx_h.at[pl.ds(wid * EMB_CPW, EMB_CPW)], vidx_v)
    for t in range(EMB_CPW):
        pltpu.async_copy(key_hbm.at[kidx_v.at[t]], bufs.at[t], sem_g)
        pltpu.async_copy(val_hbm.at[vidx_v.at[t]], bufs.at[EMB_CPW + t], sem_g)
    for t in range(2 * EMB_CPW):
        pltpu.make_async_copy(key_hbm.at[pl.ds(0, EMB_CHUNK)], bufs.at[t],
                              sem_g).wait()
    for t in range(EMB_CPW):
        base = (wid * EMB_CPW + t) * EMB_CHUNK
        pltpu.async_copy(bufs.at[t], a_h.at[pl.ds(base, EMB_CHUNK)], sem_s)
        pltpu.async_copy(bufs.at[EMB_CPW + t], b_h.at[pl.ds(base, EMB_CHUNK)],
                         sem_s)
    for t in range(2 * EMB_CPW):
        pltpu.make_async_copy(bufs.at[t], key_hbm.at[pl.ds(0, EMB_CHUNK)],
                              sem_s).wait()


@jax.jit
def _sc_emb(key_emb, val_emb, kidx, vidx):
    kern = pl.kernel(
        _sc_emb_body,
        out_type=(
            jax.ShapeDtypeStruct((N_PAD, H), jnp.float32),
            jax.ShapeDtypeStruct((N_PAD, H), jnp.float32),
        ),
        mesh=_mesh(),
        scratch_types=[
            pltpu.VMEM((EMB_CPW, EMB_CHUNK), jnp.int32),
            pltpu.VMEM((EMB_CPW, EMB_CHUNK), jnp.int32),
            pltpu.VMEM((2 * EMB_CPW, EMB_CHUNK, H), jnp.float32),
            pltpu.SemaphoreType.DMA,
            pltpu.SemaphoreType.DMA,
        ],
        compiler_params=_SC_PARAMS,
    )
    return kern(key_emb, val_emb, kidx, vidx)


def _sc_spmm_body(with_deg, *refs):
    if with_deg:
        (y_hbm, src_h, dst_h, zagg_h, zdeg_h, ones_h, aggp_h, degp_h,
         sidx_v, didx_v, ones_v, bufs, agg_sh, deg_sh,
         sem_g, sem_s, sem_d) = refs
    else:
        (y_hbm, src_h, dst_h, zagg_h, aggp_h,
         sidx_v, didx_v, bufs, agg_sh,
         sem_g, sem_s) = refs
    c = lax.axis_index("c")
    s = lax.axis_index("s")

    zbase = s * AGG_PER_TILE
    pltpu.sync_copy(zagg_h.at[pl.ds(zbase, AGG_PER_TILE)],
                    agg_sh.at[pl.ds(zbase, AGG_PER_TILE)])
    if with_deg:
        dzbase = s * DEG_PER_TILE
        pltpu.sync_copy(zdeg_h.at[pl.ds(dzbase, DEG_PER_TILE)],
                        deg_sh.at[pl.ds(dzbase, DEG_PER_TILE)])
        pltpu.sync_copy(ones_h, ones_v)

    pltpu.sync_copy(src_h.at[pl.ds(s * CPW, CPW)], sidx_v)
    pltpu.sync_copy(dst_h.at[pl.ds(s * CPW, CPW)], didx_v)

    yc = y_hbm.at[c]

    def fire_gather(g, bank):
        pltpu.async_copy(yc.at[sidx_v.at[g]], bufs.at[bank], sem_g)

    def drain(bank, sem):
        pltpu.make_async_copy(yc.at[pl.ds(0, CHUNK)], bufs.at[bank],
                              sem).wait()

    fire_gather(0, 0)
    plsc.subcore_barrier()

    def body(g, carry):
        p = lax.rem(g, 2)
        drain(p, sem_g)

        @pl.when(g > 0)
        def _():
            drain(1 - p, sem_s)

        @pl.when(g + 1 < NG)
        def _():
            fire_gather(g + 1, 1 - p)

        pltpu.async_copy(bufs.at[p], agg_sh.at[didx_v.at[g]], sem_s, add=True)
        if with_deg:
            pltpu.async_copy(ones_v, deg_sh.at[didx_v.at[g]], sem_d, add=True)
        return carry

    lax.fori_loop(0, NG, body, 0)
    drain(lax.rem(NG - 1, 2), sem_s)
    if with_deg:
        pltpu.make_async_copy(dst_h.at[pl.ds(0, CPW)], didx_v, sem_d).wait()
    plsc.subcore_barrier()

    pltpu.sync_copy(agg_sh.at[pl.ds(zbase, AGG_PER_TILE)],
                    aggp_h.at[c, pl.ds(zbase, AGG_PER_TILE)])
    if with_deg:
        pltpu.sync_copy(deg_sh.at[pl.ds(dzbase, DEG_PER_TILE)],
                        degp_h.at[c, pl.ds(dzbase, DEG_PER_TILE)])


def _make_spmm(with_deg):
    out_type = [jax.ShapeDtypeStruct((NC, AGG_ROWS, HC), jnp.float32)]
    scratch = [
        pltpu.VMEM((CPW, CHUNK), jnp.int32),
        pltpu.VMEM((CPW, CHUNK), jnp.int32),
    ]
    if with_deg:
        out_type.append(jax.ShapeDtypeStruct((NC, DEG_LEN), jnp.float32))
        scratch.append(pltpu.VMEM((CHUNK,), jnp.float32))
    scratch.append(pltpu.VMEM((2, CHUNK, HC), jnp.float32))
    scratch.append(pltpu.VMEM_SHARED((AGG_ROWS, HC), jnp.float32))
    if with_deg:
        scratch.append(pltpu.VMEM_SHARED((DEG_LEN,), jnp.float32))
    scratch.append(pltpu.SemaphoreType.DMA)
    scratch.append(pltpu.SemaphoreType.DMA)
    if with_deg:
        scratch.append(pltpu.SemaphoreType.DMA)

    return pl.kernel(
        functools.partial(_sc_spmm_body, with_deg),
        out_type=tuple(out_type),
        mesh=_mesh(),
        scratch_types=scratch,
        compiler_params=_SC_PARAMS,
    )


@jax.jit
def _sc_spmm_deg(y, src2, dst2, zagg, zdeg, ones):
    return _make_spmm(True)(y, src2, dst2, zagg, zdeg, ones)


@jax.jit
def _sc_spmm(y, src2, dst2, zagg):
    return _make_spmm(False)(y, src2, dst2, zagg)[0]


ROWS_BLK = 1000


def _tc1_body(a_ref, b_ref, ws_ref, wn_ref, s_ref, y_ref):
    h = jnp.maximum(a_ref[...] + b_ref[...], 0.0)
    s_ref[...] = jnp.dot(h, ws_ref[...], preferred_element_type=jnp.float32)
    y = jnp.dot(h, wn_ref[...], preferred_element_type=jnp.float32)
    y_ref[0] = y[:, :HC]
    y_ref[1] = y[:, HC:]


@jax.jit
def _tc1(a, b, ws, wn):
    grid = (N // ROWS_BLK,)
    blk = pl.BlockSpec((ROWS_BLK, H), lambda i: (i, 0))
    wblk = pl.BlockSpec((H, H), lambda i: (0, 0))
    yblk = pl.BlockSpec((NC, ROWS_BLK, HC), lambda i: (0, i, 0))
    return pl.pallas_call(
        _tc1_body,
        grid=grid,
        in_specs=[blk, blk, wblk, wblk],
        out_specs=[blk, yblk],
        out_shape=[
            jax.ShapeDtypeStruct((N, H), jnp.float32),
            jax.ShapeDtypeStruct((NC, N, HC), jnp.float32),
        ],
    )(a, b, ws, wn)


def _tc2_body(s0_ref, a0_ref, a1_ref, d_ref, ws_ref, wn_ref, s_ref, y_ref):
    scale = 1.0 / jnp.maximum(d_ref[0], 1.0)
    agg = jnp.concatenate([a0_ref[0], a1_ref[0]], axis=1)
    h = s0_ref[...] + agg * scale
    s_ref[...] = jnp.dot(h, ws_ref[...], preferred_element_type=jnp.float32)
    y = jnp.dot(h, wn_ref[...], preferred_element_type=jnp.float32)
    y_ref[0] = y[:, :HC]
    y_ref[1] = y[:, HC:]


@jax.jit
def _tc2(s0, aggp, degp, ws, wn):
    grid = (N // ROWS_BLK,)
    blk = pl.BlockSpec((ROWS_BLK, H), lambda i: (i, 0))
    a0blk = pl.BlockSpec((1, ROWS_BLK, HC), lambda i: (0, i, 0))
    a1blk = pl.BlockSpec((1, ROWS_BLK, HC), lambda i: (1, i, 0))
    dblk = pl.BlockSpec((1, ROWS_BLK, 1), lambda i: (0, i, 0))
    wblk = pl.BlockSpec((H, H), lambda i: (0, 0))
    yblk = pl.BlockSpec((NC, ROWS_BLK, HC), lambda i: (0, i, 0))
    return pl.pallas_call(
        _tc2_body,
        grid=grid,
        in_specs=[blk, a0blk, a1blk, dblk, wblk, wblk],
        out_specs=[blk, yblk],
        out_shape=[
            jax.ShapeDtypeStruct((N, H), jnp.float32),
            jax.ShapeDtypeStruct((NC, N, HC), jnp.float32),
        ],
    )(s0, aggp, aggp, degp, ws, wn)


def _tc3_body(s1_ref, a0_ref, a1_ref, d_ref, wc_ref, o_ref):
    scale = 1.0 / jnp.maximum(d_ref[0], 1.0)
    agg = jnp.concatenate([a0_ref[0], a1_ref[0]], axis=1)
    h = s1_ref[...] + agg * scale
    o_ref[...] = jnp.dot(h, wc_ref[...], preferred_element_type=jnp.float32)


@jax.jit
def _tc3(s1, aggp, degp, wc):
    grid = (N // ROWS_BLK,)
    blk = pl.BlockSpec((ROWS_BLK, H), lambda i: (i, 0))
    a0blk = pl.BlockSpec((1, ROWS_BLK, HC), lambda i: (0, i, 0))
    a1blk = pl.BlockSpec((1, ROWS_BLK, HC), lambda i: (1, i, 0))
    dblk = pl.BlockSpec((1, ROWS_BLK, 1), lambda i: (0, i, 0))
    wblk = pl.BlockSpec((H, OUT), lambda i: (0, 0))
    oblk = pl.BlockSpec((ROWS_BLK, OUT), lambda i: (i, 0))
    return pl.pallas_call(
        _tc3_body,
        grid=grid,
        in_specs=[blk, a0blk, a1blk, dblk, wblk],
        out_specs=oblk,
        out_shape=jax.ShapeDtypeStruct((N, OUT), jnp.float32),
    )(s1, aggp, aggp, degp, wc)


def kernel(feats, edge_index, key_emb, val_emb, W_self0, W_neigh0, W_self1,
           W_neigh1, W_cls):
    kidx = jnp.pad(feats[:, 0], (0, N_PAD - N)).reshape(-1, EMB_CHUNK)
    vidx = jnp.pad(feats[:, 1], (0, N_PAD - N)).reshape(-1, EMB_CHUNK)
    src2 = jnp.pad(edge_index[0], (0, E_PAD - E)).reshape(-1, CHUNK)
    dst2 = jnp.pad(edge_index[1], (0, E_PAD - E),
                   constant_values=N).reshape(-1, CHUNK)
    zdeg = jnp.zeros((DEG_LEN,), jnp.float32)
    zagg = jnp.zeros((AGG_ROWS, HC), jnp.float32)
    ones = jnp.ones((CHUNK,), jnp.float32)

    a_pad, b_pad = _sc_emb(key_emb, val_emb, kidx, vidx)
    s0, y0 = _tc1(a_pad, b_pad, W_self0, W_neigh0)
    aggp0, degp = _sc_spmm_deg(y0, src2, dst2, zagg, zdeg, ones)
    degp3 = degp[:1].reshape(1, DEG_LEN, 1)
    s1, y1 = _tc2(s0, aggp0, degp3, W_self1, W_neigh1)
    aggp1 = _sc_spmm(y1, src2, dst2, zagg)
    out = _tc3(s1, aggp1, degp3, W_cls)
    return out
